# baseline (speedup 1.0000x reference)
import functools

import jax
import jax.numpy as jnp
from jax import lax
from jax.experimental import pallas as pl
from jax.experimental.pallas import tpu as pltpu
from jax.experimental.pallas import tpu_sc as plsc

B = 4096
M = 1024
D = 64
NC = 2
NS = 16
NW = NC * NS
L = 16
ROWS_W = B // NW
TILE = 16
NT = ROWS_W // TILE
MC = M // L
S_ROWS = M // NS

_mesh = plsc.VectorSubcoreMesh(
    core_axis_name="c", subcore_axis_name="s", num_cores=NC, num_subcores=NS
)


@functools.partial(
    pl.kernel,
    out_type=jax.ShapeDtypeStruct((B,), jnp.float32),
    mesh=_mesh,
    compiler_params=pltpu.CompilerParams(needs_layout_passes=False),
    scratch_types=[
        pltpu.VMEM((2, TILE, M), jnp.float32),
        pltpu.VMEM((M,), jnp.float32),
        pltpu.VMEM((S_ROWS, D), jnp.float32),
        pltpu.VMEM((L * L,), jnp.float32),
        pltpu.VMEM((ROWS_W,), jnp.float32),
        pltpu.VMEM_SHARED((M,), jnp.float32),
        pltpu.SemaphoreType.DMA,
        pltpu.SemaphoreType.DMA,
    ],
)
def _sc_matvec(a_hbm, m_hbm, out_hbm, a_buf, s_buf, m_buf, tr, out_buf,
               s_shared, sem0, sem1):
    cid = lax.axis_index("c")
    sid = lax.axis_index("s")
    wid = sid * NC + cid
    base = wid * ROWS_W
    iota = lax.iota(jnp.int32, L)
    sems = (sem0, sem1)

    def a_copy(g, slot):
        return pltpu.make_async_copy(
            a_hbm.at[pl.ds(base + g * TILE, TILE), :],
            a_buf.at[slot],
            sems[slot],
        )

    a_copy(0, 0).start()
    a_copy(1, 1).start()

    pltpu.sync_copy(m_hbm.at[pl.ds(sid * S_ROWS, S_ROWS), :], m_buf)
    for j in range(S_ROWS // L):
        for r in range(L):
            row = j * L + r
            p = (m_buf[row, pl.ds(0, L)] + m_buf[row, pl.ds(L, L)]
                 + m_buf[row, pl.ds(2 * L, L)] + m_buf[row, pl.ds(3 * L, L)])
            tr[pl.ds(r * L, L)] = p
        res = jnp.zeros((L,), jnp.float32)
        for c in range(L):
            res = res + plsc.load_gather(tr, [iota * L + c])
        s_buf[pl.ds(sid * S_ROWS + j * L, L)] = res * (1.0 / D)
    pltpu.sync_copy(s_buf.at[pl.ds(sid * S_ROWS, S_ROWS)],
                    s_shared.at[pl.ds(sid * S_ROWS, S_ROWS)])
    plsc.subcore_barrier()
    pltpu.sync_copy(s_shared, s_buf)

    for g in range(NT):
        slot = g & 1
        a_copy(g, slot).wait()

        def mac(c, accs, _slot=slot):
            off = c * L
            vc = s_buf[pl.ds(off, L)]
            return tuple(
                accs[r] + a_buf[_slot, r, pl.ds(off, L)] * vc
                for r in range(L)
            )

        accs = lax.fori_loop(
            0, MC, mac, tuple(jnp.zeros((L,), jnp.float32) for _ in range(L)))
        for r in range(L):
            tr[pl.ds(r * L, L)] = accs[r]
        res = jnp.zeros((L,), jnp.float32)
        for c in range(L):
            res = res + plsc.load_gather(tr, [iota * L + c])
        out_buf[pl.ds(g * TILE, L)] = res
        if g + 2 < NT:
            a_copy(g + 2, slot).start()

    pltpu.sync_copy(out_buf, out_hbm.at[pl.ds(base, ROWS_W)])


@jax.jit
def kernel(attention_weights, ability_means):
    return _sc_matvec(attention_weights, ability_means)
